# Initial kernel scaffold; baseline (speedup 1.0000x reference)
import jax
import jax.numpy as jnp
from jax.experimental import pallas as pl


def kernel(x, edge_index, Wl1, Wr1, b1, Wl2, Wr2, b2, W_fc1, b_fc1, W_fc2, b_fc2):
    raise NotImplementedError("write your pallas kernel here")


# speedup vs baseline: 12.3899x; 12.3899x over previous
import functools

import jax
import jax.numpy as jnp
from jax import lax
from jax.experimental import pallas as pl
from jax.experimental.pallas import tpu as pltpu
from jax.experimental.pallas import tpu_sc as plsc

N = 10000
NPAD = 10240
E = 320000
DPAD = 80

NC = 2
NS = 16
E_PER_TILE = E // (NC * NS)
N_PER_TILE = NPAD // NS

K1 = 400
K2 = 2000
K3 = 2000


def _tc_a_body(x_ref, wl_ref, wr_ref, b_ref, y1aug_ref, hr_ref):
    xb = x_ref[...]
    y1 = jnp.dot(xb, wl_ref[...], preferred_element_type=jnp.float32)
    r = xb.shape[0]
    ones = jnp.ones((r, 1), jnp.float32)
    zeros = jnp.zeros((r, DPAD - 65), jnp.float32)
    y1aug_ref[...] = jnp.concatenate([y1, ones, zeros], axis=1)
    hr_ref[...] = jnp.dot(xb, wr_ref[...], preferred_element_type=jnp.float32) + b_ref[...]


def _tc_a(x, wl1, wr1, b1):
    blk = 1000
    grid = N // blk
    return pl.pallas_call(
        _tc_a_body,
        grid=(grid,),
        in_specs=[
            pl.BlockSpec((blk, 128), lambda i: (i, 0)),
            pl.BlockSpec((128, 64), lambda i: (0, 0)),
            pl.BlockSpec((128, 64), lambda i: (0, 0)),
            pl.BlockSpec((1, 64), lambda i: (0, 0)),
        ],
        out_specs=[
            pl.BlockSpec((blk, DPAD), lambda i: (i, 0)),
            pl.BlockSpec((blk, 64), lambda i: (i, 0)),
        ],
        out_shape=[
            jax.ShapeDtypeStruct((N, DPAD), jnp.float32),
            jax.ShapeDtypeStruct((N, 64), jnp.float32),
        ],
    )(x, wl1, wr1, b1)


def _tc_b_body(acc_ref, hr_ref, wl2_ref, wr2_ref, b2_ref, y2_ref, zr_ref, inv_ref):
    s = acc_ref[0] + acc_ref[1]
    cnt = s[:, 64:65]
    inv = 1.0 / jnp.maximum(cnt, 1.0)
    h = s[:, :64] * inv + hr_ref[...]
    y2_ref[...] = jnp.dot(h, wl2_ref[...], preferred_element_type=jnp.float32)
    zr_ref[...] = jnp.dot(h, wr2_ref[...], preferred_element_type=jnp.float32) + b2_ref[...]
    inv_ref[...] = inv


def _tc_b(acc1, hr, wl2, wr2, b2):
    blk = 1000
    grid = N // blk
    return pl.pallas_call(
        _tc_b_body,
        grid=(grid,),
        in_specs=[
            pl.BlockSpec((2, blk, DPAD), lambda i: (0, i, 0)),
            pl.BlockSpec((blk, 64), lambda i: (i, 0)),
            pl.BlockSpec((64, 16), lambda i: (0, 0)),
            pl.BlockSpec((64, 16), lambda i: (0, 0)),
            pl.BlockSpec((1, 16), lambda i: (0, 0)),
        ],
        out_specs=[
            pl.BlockSpec((blk, 16), lambda i: (i, 0)),
            pl.BlockSpec((blk, 16), lambda i: (i, 0)),
            pl.BlockSpec((blk, 1), lambda i: (i, 0)),
        ],
        out_shape=[
            jax.ShapeDtypeStruct((N, 16), jnp.float32),
            jax.ShapeDtypeStruct((N, 16), jnp.float32),
            jax.ShapeDtypeStruct((N, 1), jnp.float32),
        ],
    )(acc1, hr, wl2, wr2, b2)


def _tc_c_body(acc2_ref, inv_ref, zr_ref, wf2t_ref, wf1t_ref, z_ref, zw_ref):
    s = acc2_ref[0] + acc2_ref[1]
    z = s * inv_ref[...] + zr_ref[...]
    z_ref[...] = z
    wrow = jnp.dot(wf2t_ref[...], wf1t_ref[...], preferred_element_type=jnp.float32)
    zw_ref[...] = z * wrow


def _tc_c(acc2, inv, zr, wf2t, wf1t):
    blk = 1000
    grid = N // blk
    return pl.pallas_call(
        _tc_c_body,
        grid=(grid,),
        in_specs=[
            pl.BlockSpec((2, blk, 16), lambda i: (0, i, 0)),
            pl.BlockSpec((blk, 1), lambda i: (i, 0)),
            pl.BlockSpec((blk, 16), lambda i: (i, 0)),
            pl.BlockSpec((1, 8), lambda i: (0, 0)),
            pl.BlockSpec((8, 16), lambda i: (0, 0)),
        ],
        out_specs=[
            pl.BlockSpec((blk, 16), lambda i: (i, 0)),
            pl.BlockSpec((blk, 16), lambda i: (i, 0)),
        ],
        out_shape=[
            jax.ShapeDtypeStruct((N, 16), jnp.float32),
            jax.ShapeDtypeStruct((N, 16), jnp.float32),
        ],
    )(acc2, inv, zr, wf2t, wf1t)


def _make_sc_agg(dcols, kchunk):
    nchunks = E_PER_TILE // kchunk
    mesh = plsc.VectorSubcoreMesh(core_axis_name="c", subcore_axis_name="s")

    @functools.partial(
        pl.kernel,
        mesh=mesh,
        compiler_params=pltpu.CompilerParams(use_tc_tiling_on_sc=False),
        out_type=jax.ShapeDtypeStruct((NC, NPAD, dcols), jnp.float32),
        scratch_types=[
            pltpu.VMEM((kchunk,), jnp.int32),
            pltpu.VMEM((kchunk,), jnp.int32),
            pltpu.VMEM((kchunk, dcols), jnp.float32),
            pltpu.VMEM_SHARED((NPAD, dcols), jnp.float32),
            pltpu.SemaphoreType.DMA,
        ],
    )
    def agg(table, srcs, dsts, zrows, acc_out, sidx, didx, rows, acc_sh, sem):
        c = lax.axis_index("c")
        s = lax.axis_index("s")
        pltpu.sync_copy(zrows, acc_sh.at[pl.ds(s * N_PER_TILE, N_PER_TILE)])
        plsc.subcore_barrier()
        base = c * (E // NC) + s * E_PER_TILE

        def chunk(j, carry):
            off = base + j * kchunk
            pltpu.sync_copy(srcs.at[pl.ds(off, kchunk)], sidx)
            pltpu.async_copy(table.at[sidx], rows, sem).wait()
            pltpu.sync_copy(dsts.at[pl.ds(off, kchunk)], didx)
            pltpu.sync_copy(rows, acc_sh.at[didx], add=True)
            return carry

        lax.fori_loop(0, nchunks, chunk, 0)
        plsc.subcore_barrier()
        pltpu.sync_copy(
            acc_sh.at[pl.ds(s * N_PER_TILE, N_PER_TILE)],
            acc_out.at[c, pl.ds(s * N_PER_TILE, N_PER_TILE)],
        )

    return agg


_sc_agg1 = _make_sc_agg(DPAD, K1)
_sc_agg2 = _make_sc_agg(16, K2)


def _make_sc_head():
    nchunks = E_PER_TILE // K3
    ngroups = K3 // 16
    mesh = plsc.VectorSubcoreMesh(core_axis_name="c", subcore_axis_name="s")

    @functools.partial(
        pl.kernel,
        mesh=mesh,
        compiler_params=pltpu.CompilerParams(
            use_tc_tiling_on_sc=False, needs_layout_passes=False),
        out_type=jax.ShapeDtypeStruct((E,), jnp.float32),
        scratch_types=[
            pltpu.VMEM((K3,), jnp.int32),
            pltpu.VMEM((K3,), jnp.int32),
            pltpu.VMEM((K3, 16), jnp.float32),
            pltpu.VMEM((K3, 16), jnp.float32),
            pltpu.VMEM((K3,), jnp.float32),
            pltpu.VMEM((16,), jnp.float32),
            pltpu.SemaphoreType.DMA,
        ],
    )
    def head(z, zw, srcs, dsts, cvec, out, sidx, didx, zs, zd, ov, cv, sem):
        c = lax.axis_index("c")
        s = lax.axis_index("s")
        base = c * (E // NC) + s * E_PER_TILE
        pltpu.sync_copy(cvec, cv)
        cval = cv[...]
        lanes = lax.iota(jnp.int32, 16)

        def chunk(j, carry):
            off = base + j * K3
            pltpu.sync_copy(srcs.at[pl.ds(off, K3)], sidx)
            pltpu.sync_copy(dsts.at[pl.ds(off, K3)], didx)
            pltpu.async_copy(z.at[sidx], zs, sem).wait()
            pltpu.async_copy(zw.at[didx], zd, sem).wait()

            def group(i, carry2):
                rows = i * 16 + lanes
                acc = cval
                for d in range(16):
                    cols = jnp.full((16,), d, jnp.int32)
                    sv = plsc.load_gather(zs, [rows, cols])
                    dv = plsc.load_gather(zd, [rows, cols])
                    acc = acc + sv * dv
                sig = 1.0 / (1.0 + jnp.exp(-acc))
                ov[pl.ds(i * 16, 16)] = sig
                return carry2

            lax.fori_loop(0, ngroups, group, 0)
            pltpu.sync_copy(ov, out.at[pl.ds(off, K3)])
            return carry

        lax.fori_loop(0, nchunks, chunk, 0)

    return head


_sc_head = _make_sc_head()


def kernel(x, edge_index, Wl1, Wr1, b1, Wl2, Wr2, b2, W_fc1, b_fc1, W_fc2, b_fc2):
    src = edge_index[0]
    dst = edge_index[1]

    y1aug, hr = _tc_a(x, Wl1, Wr1, b1.reshape(1, 64))
    zrows1 = jnp.zeros((N_PER_TILE, DPAD), jnp.float32)
    acc1 = _sc_agg1(y1aug, src, dst, zrows1)

    y2, zr, inv = _tc_b(acc1, hr, Wl2, Wr2, b2.reshape(1, 16))
    zrows2 = jnp.zeros((N_PER_TILE, 16), jnp.float32)
    acc2 = _sc_agg2(y2, src, dst, zrows2)

    z, zw = _tc_c(acc2, inv, zr, W_fc2.T, W_fc1.T)

    cscalar = jnp.dot(b_fc1, W_fc2[:, 0]) + b_fc2[0]
    cvec = jnp.full((16,), cscalar, jnp.float32)
    out = _sc_head(z, zw, src, dst, cvec)
    return out.reshape(E, 1)


# speedup vs baseline: 15.0385x; 1.2138x over previous
import functools

import jax
import jax.numpy as jnp
from jax import lax
from jax.experimental import pallas as pl
from jax.experimental.pallas import tpu as pltpu
from jax.experimental.pallas import tpu_sc as plsc

N = 10000
NPAD = 10240
E = 320000
DPAD = 80

NC = 2
NS = 16
E_PER_TILE = E // (NC * NS)
N_PER_TILE = NPAD // NS

K1 = 200
K2 = 1000
K3 = 400


def _tc_a_body(x_ref, wl_ref, wr_ref, b_ref, y1aug_ref, hr_ref):
    xb = x_ref[...]
    y1 = jnp.dot(xb, wl_ref[...], preferred_element_type=jnp.float32)
    r = xb.shape[0]
    ones = jnp.ones((r, 1), jnp.float32)
    zeros = jnp.zeros((r, DPAD - 65), jnp.float32)
    y1aug_ref[...] = jnp.concatenate([y1, ones, zeros], axis=1)
    hr_ref[...] = jnp.dot(xb, wr_ref[...], preferred_element_type=jnp.float32) + b_ref[...]


def _tc_a(x, wl1, wr1, b1):
    blk = 1000
    grid = N // blk
    return pl.pallas_call(
        _tc_a_body,
        grid=(grid,),
        in_specs=[
            pl.BlockSpec((blk, 128), lambda i: (i, 0)),
            pl.BlockSpec((128, 64), lambda i: (0, 0)),
            pl.BlockSpec((128, 64), lambda i: (0, 0)),
            pl.BlockSpec((1, 64), lambda i: (0, 0)),
        ],
        out_specs=[
            pl.BlockSpec((blk, DPAD), lambda i: (i, 0)),
            pl.BlockSpec((blk, 64), lambda i: (i, 0)),
        ],
        out_shape=[
            jax.ShapeDtypeStruct((N, DPAD), jnp.float32),
            jax.ShapeDtypeStruct((N, 64), jnp.float32),
        ],
    )(x, wl1, wr1, b1)


def _tc_b_body(acc_ref, hr_ref, wl2_ref, wr2_ref, b2_ref, y2_ref, zr_ref, inv_ref):
    s = acc_ref[0] + acc_ref[1]
    cnt = s[:, 64:65]
    inv = 1.0 / jnp.maximum(cnt, 1.0)
    h = s[:, :64] * inv + hr_ref[...]
    y2_ref[...] = jnp.dot(h, wl2_ref[...], preferred_element_type=jnp.float32)
    zr_ref[...] = jnp.dot(h, wr2_ref[...], preferred_element_type=jnp.float32) + b2_ref[...]
    inv_ref[...] = inv


def _tc_b(acc1, hr, wl2, wr2, b2):
    blk = 1000
    grid = N // blk
    return pl.pallas_call(
        _tc_b_body,
        grid=(grid,),
        in_specs=[
            pl.BlockSpec((2, blk, DPAD), lambda i: (0, i, 0)),
            pl.BlockSpec((blk, 64), lambda i: (i, 0)),
            pl.BlockSpec((64, 16), lambda i: (0, 0)),
            pl.BlockSpec((64, 16), lambda i: (0, 0)),
            pl.BlockSpec((1, 16), lambda i: (0, 0)),
        ],
        out_specs=[
            pl.BlockSpec((blk, 16), lambda i: (i, 0)),
            pl.BlockSpec((blk, 16), lambda i: (i, 0)),
            pl.BlockSpec((blk, 1), lambda i: (i, 0)),
        ],
        out_shape=[
            jax.ShapeDtypeStruct((N, 16), jnp.float32),
            jax.ShapeDtypeStruct((N, 16), jnp.float32),
            jax.ShapeDtypeStruct((N, 1), jnp.float32),
        ],
    )(acc1, hr, wl2, wr2, b2)


def _tc_c_body(acc2_ref, inv_ref, zr_ref, wf2t_ref, wf1t_ref, z_ref, zw_ref):
    s = acc2_ref[0] + acc2_ref[1]
    z = s * inv_ref[...] + zr_ref[...]
    z_ref[...] = z
    wrow = jnp.dot(wf2t_ref[...], wf1t_ref[...], preferred_element_type=jnp.float32)
    zw_ref[...] = z * wrow


def _tc_c(acc2, inv, zr, wf2t, wf1t):
    blk = 1000
    grid = N // blk
    return pl.pallas_call(
        _tc_c_body,
        grid=(grid,),
        in_specs=[
            pl.BlockSpec((2, blk, 16), lambda i: (0, i, 0)),
            pl.BlockSpec((blk, 1), lambda i: (i, 0)),
            pl.BlockSpec((blk, 16), lambda i: (i, 0)),
            pl.BlockSpec((1, 8), lambda i: (0, 0)),
            pl.BlockSpec((8, 16), lambda i: (0, 0)),
        ],
        out_specs=[
            pl.BlockSpec((blk, 16), lambda i: (i, 0)),
            pl.BlockSpec((blk, 16), lambda i: (i, 0)),
        ],
        out_shape=[
            jax.ShapeDtypeStruct((N, 16), jnp.float32),
            jax.ShapeDtypeStruct((N, 16), jnp.float32),
        ],
    )(acc2, inv, zr, wf2t, wf1t)


def _make_sc_agg(dcols, kchunk):
    nchunks = E_PER_TILE // kchunk
    NB = 3
    NI = 4
    mesh = plsc.VectorSubcoreMesh(core_axis_name="c", subcore_axis_name="s")

    scratch = (
        [pltpu.VMEM((kchunk,), jnp.int32) for _ in range(NI)]
        + [pltpu.VMEM((kchunk,), jnp.int32) for _ in range(NI)]
        + [pltpu.VMEM((kchunk, dcols), jnp.float32) for _ in range(NB)]
        + [pltpu.VMEM_SHARED((NPAD, dcols), jnp.float32)]
        + [pltpu.SemaphoreType.DMA for _ in range(NI + 2 * NB)]
    )

    @functools.partial(
        pl.kernel,
        mesh=mesh,
        compiler_params=pltpu.CompilerParams(use_tc_tiling_on_sc=False),
        out_type=jax.ShapeDtypeStruct((NC, NPAD, dcols), jnp.float32),
        scratch_types=scratch,
    )
    def agg(table, srcs, dsts, zrows, acc_out, *sc):
        sidx = sc[0:NI]
        didx = sc[NI:2 * NI]
        rows = sc[2 * NI:2 * NI + NB]
        acc_sh = sc[2 * NI + NB]
        sem_i = sc[2 * NI + NB + 1:2 * NI + NB + 1 + NI]
        sem_g = sc[2 * NI + NB + 1 + NI:2 * NI + NB + 1 + NI + NB]
        sem_s = sc[2 * NI + NB + 1 + NI + NB:2 * NI + NB + 1 + NI + 2 * NB]

        c = lax.axis_index("c")
        s = lax.axis_index("s")
        pltpu.sync_copy(zrows, acc_sh.at[pl.ds(s * N_PER_TILE, N_PER_TILE)])
        plsc.subcore_barrier()
        base = c * (E // NC) + s * E_PER_TILE

        idx_d = {}
        gat_d = {}
        sca_d = {}

        def start_idx(j):
            b = j % NI
            off = base + j * kchunk
            idx_d[j] = (
                pltpu.async_copy(srcs.at[pl.ds(off, kchunk)], sidx[b], sem_i[b]),
                pltpu.async_copy(dsts.at[pl.ds(off, kchunk)], didx[b], sem_i[b]),
            )

        def start_gather(j):
            gat_d[j] = pltpu.async_copy(
                table.at[sidx[j % NI]], rows[j % NB], sem_g[j % NB])

        def start_scatter(j):
            sca_d[j] = pltpu.async_copy(
                rows[j % NB], acc_sh.at[didx[j % NI]], sem_s[j % NB], add=True)

        start_idx(0)
        if nchunks > 1:
            start_idx(1)
        idx_d[0][0].wait()
        idx_d[0][1].wait()
        start_gather(0)
        for j in range(nchunks):
            gat_d[j].wait()
            if j + 1 < nchunks:
                idx_d[j + 1][0].wait()
                idx_d[j + 1][1].wait()
                if j + 1 >= NB:
                    sca_d[j + 1 - NB].wait()
                start_gather(j + 1)
            start_scatter(j)
            if j + 2 < nchunks:
                start_idx(j + 2)
        for j in range(max(0, nchunks - NB), nchunks):
            sca_d[j].wait()

        plsc.subcore_barrier()
        pltpu.sync_copy(
            acc_sh.at[pl.ds(s * N_PER_TILE, N_PER_TILE)],
            acc_out.at[c, pl.ds(s * N_PER_TILE, N_PER_TILE)],
        )

    return agg


_sc_agg1 = _make_sc_agg(DPAD, K1)
_sc_agg2 = _make_sc_agg(16, K2)


def _make_sc_head():
    nchunks = E_PER_TILE // K3
    ngroups = K3 // 16
    NI = 3
    mesh = plsc.VectorSubcoreMesh(core_axis_name="c", subcore_axis_name="s")

    scratch = (
        [pltpu.VMEM((K3,), jnp.int32) for _ in range(NI)]
        + [pltpu.VMEM((K3,), jnp.int32) for _ in range(NI)]
        + [pltpu.VMEM((K3, 16), jnp.float32) for _ in range(2)]
        + [pltpu.VMEM((K3, 16), jnp.float32) for _ in range(2)]
        + [pltpu.VMEM((K3,), jnp.float32) for _ in range(2)]
        + [pltpu.VMEM((16,), jnp.float32)]
        + [pltpu.SemaphoreType.DMA for _ in range(NI + 6)]
    )

    @functools.partial(
        pl.kernel,
        mesh=mesh,
        compiler_params=pltpu.CompilerParams(
            use_tc_tiling_on_sc=False, needs_layout_passes=False),
        out_type=jax.ShapeDtypeStruct((E,), jnp.float32),
        scratch_types=scratch,
    )
    def head(z, zw, srcs, dsts, cvec, out, *sc):
        sidx = sc[0:NI]
        didx = sc[NI:2 * NI]
        zs = sc[2 * NI:2 * NI + 2]
        zd = sc[2 * NI + 2:2 * NI + 4]
        ov = sc[2 * NI + 4:2 * NI + 6]
        cv = sc[2 * NI + 6]
        sem_i = sc[2 * NI + 7:2 * NI + 7 + NI]
        sem_zs = sc[2 * NI + 7 + NI:2 * NI + 9 + NI]
        sem_zd = sc[2 * NI + 9 + NI:2 * NI + 11 + NI]
        sem_o = sc[2 * NI + 11 + NI:2 * NI + 13 + NI]

        c = lax.axis_index("c")
        s = lax.axis_index("s")
        base = c * (E // NC) + s * E_PER_TILE
        pltpu.sync_copy(cvec, cv)
        cval = cv[...]
        lanes = lax.iota(jnp.int32, 16)

        idx_d = {}
        gat_d = {}
        out_d = {}

        def start_idx(j):
            b = j % NI
            off = base + j * K3
            idx_d[j] = (
                pltpu.async_copy(srcs.at[pl.ds(off, K3)], sidx[b], sem_i[b]),
                pltpu.async_copy(dsts.at[pl.ds(off, K3)], didx[b], sem_i[b]),
            )

        def start_gathers(j):
            b = j % 2
            gat_d[j] = (
                pltpu.async_copy(z.at[sidx[j % NI]], zs[b], sem_zs[b]),
                pltpu.async_copy(zw.at[didx[j % NI]], zd[b], sem_zd[b]),
            )

        start_idx(0)
        if nchunks > 1:
            start_idx(1)
        idx_d[0][0].wait()
        idx_d[0][1].wait()
        start_gathers(0)
        for j in range(nchunks):
            b = j % 2
            gat_d[j][0].wait()
            gat_d[j][1].wait()
            if j + 1 < nchunks:
                idx_d[j + 1][0].wait()
                idx_d[j + 1][1].wait()
                start_gathers(j + 1)
            if j + 2 < nchunks:
                start_idx(j + 2)
            if j >= 2:
                out_d[j - 2].wait()

            zsb = zs[b]
            zdb = zd[b]
            ovb = ov[b]

            def group(i, carry2):
                rows = i * 16 + lanes
                acc = cval
                for d in range(16):
                    cols = jnp.full((16,), d, jnp.int32)
                    sv = plsc.load_gather(zsb, [rows, cols])
                    dv = plsc.load_gather(zdb, [rows, cols])
                    acc = acc + sv * dv
                sig = 1.0 / (1.0 + jnp.exp(-acc))
                ovb[pl.ds(i * 16, 16)] = sig
                return carry2

            lax.fori_loop(0, ngroups, group, 0)
            out_d[j] = pltpu.async_copy(
                ovb, out.at[pl.ds(base + j * K3, K3)], sem_o[b])
        for j in range(max(0, nchunks - 2), nchunks):
            out_d[j].wait()

    return head


_sc_head = _make_sc_head()


def kernel(x, edge_index, Wl1, Wr1, b1, Wl2, Wr2, b2, W_fc1, b_fc1, W_fc2, b_fc2):
    src = edge_index[0]
    dst = edge_index[1]

    y1aug, hr = _tc_a(x, Wl1, Wr1, b1.reshape(1, 64))
    zrows1 = jnp.zeros((N_PER_TILE, DPAD), jnp.float32)
    acc1 = _sc_agg1(y1aug, src, dst, zrows1)

    y2, zr, inv = _tc_b(acc1, hr, Wl2, Wr2, b2.reshape(1, 16))
    zrows2 = jnp.zeros((N_PER_TILE, 16), jnp.float32)
    acc2 = _sc_agg2(y2, src, dst, zrows2)

    z, zw = _tc_c(acc2, inv, zr, W_fc2.T, W_fc1.T)

    cscalar = jnp.dot(b_fc1, W_fc2[:, 0]) + b_fc2[0]
    cvec = jnp.full((16,), cscalar, jnp.float32)
    out = _sc_head(z, zw, src, dst, cvec)
    return out.reshape(E, 1)


# speedup vs baseline: 15.0758x; 1.0025x over previous
import functools

import jax
import jax.numpy as jnp
from jax import lax
from jax.experimental import pallas as pl
from jax.experimental.pallas import tpu as pltpu
from jax.experimental.pallas import tpu_sc as plsc

N = 10000
NPAD = 10240
E = 320000
DPAD = 80

NC = 2
NS = 16
E_PER_TILE = E // (NC * NS)
N_PER_TILE = NPAD // NS

K1 = 200
K2 = 1000
K3 = 400


def _tc_a_body(x_ref, wl_ref, wr_ref, b_ref, y1aug_ref, hr_ref):
    xb = x_ref[...]
    y1 = jnp.dot(xb, wl_ref[...], preferred_element_type=jnp.float32)
    r = xb.shape[0]
    ones = jnp.ones((r, 1), jnp.float32)
    zeros = jnp.zeros((r, DPAD - 65), jnp.float32)
    y1aug_ref[...] = jnp.concatenate([y1, ones, zeros], axis=1)
    hr_ref[...] = jnp.dot(xb, wr_ref[...], preferred_element_type=jnp.float32) + b_ref[...]


def _tc_a(x, wl1, wr1, b1):
    blk = 1000
    grid = N // blk
    return pl.pallas_call(
        _tc_a_body,
        grid=(grid,),
        in_specs=[
            pl.BlockSpec((blk, 128), lambda i: (i, 0)),
            pl.BlockSpec((128, 64), lambda i: (0, 0)),
            pl.BlockSpec((128, 64), lambda i: (0, 0)),
            pl.BlockSpec((1, 64), lambda i: (0, 0)),
        ],
        out_specs=[
            pl.BlockSpec((blk, DPAD), lambda i: (i, 0)),
            pl.BlockSpec((blk, 64), lambda i: (i, 0)),
        ],
        out_shape=[
            jax.ShapeDtypeStruct((N, DPAD), jnp.float32),
            jax.ShapeDtypeStruct((N, 64), jnp.float32),
        ],
    )(x, wl1, wr1, b1)


def _tc_b_body(acc_ref, hr_ref, wl2_ref, wr2_ref, b2_ref, y2_ref, zr_ref, inv_ref):
    s = acc_ref[0] + acc_ref[1]
    cnt = s[:, 64:65]
    inv = 1.0 / jnp.maximum(cnt, 1.0)
    h = s[:, :64] * inv + hr_ref[...]
    y2_ref[...] = jnp.dot(h, wl2_ref[...], preferred_element_type=jnp.float32)
    zr_ref[...] = jnp.dot(h, wr2_ref[...], preferred_element_type=jnp.float32) + b2_ref[...]
    inv_ref[...] = inv


def _tc_b(acc1, hr, wl2, wr2, b2):
    blk = 1000
    grid = N // blk
    return pl.pallas_call(
        _tc_b_body,
        grid=(grid,),
        in_specs=[
            pl.BlockSpec((2, blk, DPAD), lambda i: (0, i, 0)),
            pl.BlockSpec((blk, 64), lambda i: (i, 0)),
            pl.BlockSpec((64, 16), lambda i: (0, 0)),
            pl.BlockSpec((64, 16), lambda i: (0, 0)),
            pl.BlockSpec((1, 16), lambda i: (0, 0)),
        ],
        out_specs=[
            pl.BlockSpec((blk, 16), lambda i: (i, 0)),
            pl.BlockSpec((blk, 16), lambda i: (i, 0)),
            pl.BlockSpec((blk, 1), lambda i: (i, 0)),
        ],
        out_shape=[
            jax.ShapeDtypeStruct((N, 16), jnp.float32),
            jax.ShapeDtypeStruct((N, 16), jnp.float32),
            jax.ShapeDtypeStruct((N, 1), jnp.float32),
        ],
    )(acc1, hr, wl2, wr2, b2)


def _tc_c_body(acc2_ref, inv_ref, zr_ref, wf2t_ref, wf1t_ref, z_ref, zw_ref):
    s = acc2_ref[0] + acc2_ref[1]
    z = s * inv_ref[...] + zr_ref[...]
    z_ref[...] = z
    wrow = jnp.dot(wf2t_ref[...], wf1t_ref[...], preferred_element_type=jnp.float32)
    zw_ref[...] = z * wrow


def _tc_c(acc2, inv, zr, wf2t, wf1t):
    blk = 1000
    grid = N // blk
    return pl.pallas_call(
        _tc_c_body,
        grid=(grid,),
        in_specs=[
            pl.BlockSpec((2, blk, 16), lambda i: (0, i, 0)),
            pl.BlockSpec((blk, 1), lambda i: (i, 0)),
            pl.BlockSpec((blk, 16), lambda i: (i, 0)),
            pl.BlockSpec((1, 8), lambda i: (0, 0)),
            pl.BlockSpec((8, 16), lambda i: (0, 0)),
        ],
        out_specs=[
            pl.BlockSpec((blk, 16), lambda i: (i, 0)),
            pl.BlockSpec((blk, 16), lambda i: (i, 0)),
        ],
        out_shape=[
            jax.ShapeDtypeStruct((N, 16), jnp.float32),
            jax.ShapeDtypeStruct((N, 16), jnp.float32),
        ],
    )(acc2, inv, zr, wf2t, wf1t)


def _make_sc_agg(dcols, kchunk):
    nchunks = E_PER_TILE // kchunk
    NB = 4
    NI = 6
    mesh = plsc.VectorSubcoreMesh(core_axis_name="c", subcore_axis_name="s")

    scratch = (
        [pltpu.VMEM((kchunk,), jnp.int32) for _ in range(NI)]
        + [pltpu.VMEM((kchunk,), jnp.int32) for _ in range(NI)]
        + [pltpu.VMEM((kchunk, dcols), jnp.float32) for _ in range(NB)]
        + [pltpu.VMEM_SHARED((NPAD, dcols), jnp.float32)]
        + [pltpu.SemaphoreType.DMA for _ in range(NI + 2 * NB)]
    )

    @functools.partial(
        pl.kernel,
        mesh=mesh,
        compiler_params=pltpu.CompilerParams(use_tc_tiling_on_sc=False),
        out_type=jax.ShapeDtypeStruct((NC, NPAD, dcols), jnp.float32),
        scratch_types=scratch,
    )
    def agg(table, srcs, dsts, zrows, acc_out, *sc):
        sidx = sc[0:NI]
        didx = sc[NI:2 * NI]
        rows = sc[2 * NI:2 * NI + NB]
        acc_sh = sc[2 * NI + NB]
        sem_i = sc[2 * NI + NB + 1:2 * NI + NB + 1 + NI]
        sem_g = sc[2 * NI + NB + 1 + NI:2 * NI + NB + 1 + NI + NB]
        sem_s = sc[2 * NI + NB + 1 + NI + NB:2 * NI + NB + 1 + NI + 2 * NB]

        c = lax.axis_index("c")
        s = lax.axis_index("s")
        pltpu.sync_copy(zrows, acc_sh.at[pl.ds(s * N_PER_TILE, N_PER_TILE)])
        plsc.subcore_barrier()
        base = c * (E // NC) + s * E_PER_TILE

        idx_d = {}
        gat_d = {}
        sca_d = {}

        def start_idx(j):
            b = j % NI
            off = base + j * kchunk
            idx_d[j] = (
                pltpu.async_copy(srcs.at[pl.ds(off, kchunk)], sidx[b], sem_i[b]),
                pltpu.async_copy(dsts.at[pl.ds(off, kchunk)], didx[b], sem_i[b]),
            )

        def start_gather(j):
            gat_d[j] = pltpu.async_copy(
                table.at[sidx[j % NI]], rows[j % NB], sem_g[j % NB])

        def start_scatter(j):
            sca_d[j] = pltpu.async_copy(
                rows[j % NB], acc_sh.at[didx[j % NI]], sem_s[j % NB], add=True)

        start_idx(0)
        if nchunks > 1:
            start_idx(1)
        idx_d[0][0].wait()
        idx_d[0][1].wait()
        start_gather(0)
        for j in range(nchunks):
            gat_d[j].wait()
            if j + 1 < nchunks:
                idx_d[j + 1][0].wait()
                idx_d[j + 1][1].wait()
                if j + 1 >= NB:
                    sca_d[j + 1 - NB].wait()
                start_gather(j + 1)
            start_scatter(j)
            if j + 2 < nchunks:
                start_idx(j + 2)
        for j in range(max(0, nchunks - NB), nchunks):
            sca_d[j].wait()

        plsc.subcore_barrier()
        pltpu.sync_copy(
            acc_sh.at[pl.ds(s * N_PER_TILE, N_PER_TILE)],
            acc_out.at[c, pl.ds(s * N_PER_TILE, N_PER_TILE)],
        )

    return agg


_sc_agg1 = _make_sc_agg(DPAD, K1)
_sc_agg2 = _make_sc_agg(16, K2)


def _make_sc_head():
    nchunks = E_PER_TILE // K3
    ngroups = K3 // 16
    NI = 3
    mesh = plsc.VectorSubcoreMesh(core_axis_name="c", subcore_axis_name="s")

    scratch = (
        [pltpu.VMEM((K3,), jnp.int32) for _ in range(NI)]
        + [pltpu.VMEM((K3,), jnp.int32) for _ in range(NI)]
        + [pltpu.VMEM((K3, 16), jnp.float32) for _ in range(2)]
        + [pltpu.VMEM((K3, 16), jnp.float32) for _ in range(2)]
        + [pltpu.VMEM((K3,), jnp.float32) for _ in range(2)]
        + [pltpu.VMEM((16,), jnp.float32)]
        + [pltpu.SemaphoreType.DMA for _ in range(NI + 6)]
    )

    @functools.partial(
        pl.kernel,
        mesh=mesh,
        compiler_params=pltpu.CompilerParams(
            use_tc_tiling_on_sc=False, needs_layout_passes=False),
        out_type=jax.ShapeDtypeStruct((E,), jnp.float32),
        scratch_types=scratch,
    )
    def head(z, zw, srcs, dsts, cvec, out, *sc):
        sidx = sc[0:NI]
        didx = sc[NI:2 * NI]
        zs = sc[2 * NI:2 * NI + 2]
        zd = sc[2 * NI + 2:2 * NI + 4]
        ov = sc[2 * NI + 4:2 * NI + 6]
        cv = sc[2 * NI + 6]
        sem_i = sc[2 * NI + 7:2 * NI + 7 + NI]
        sem_zs = sc[2 * NI + 7 + NI:2 * NI + 9 + NI]
        sem_zd = sc[2 * NI + 9 + NI:2 * NI + 11 + NI]
        sem_o = sc[2 * NI + 11 + NI:2 * NI + 13 + NI]

        c = lax.axis_index("c")
        s = lax.axis_index("s")
        base = c * (E // NC) + s * E_PER_TILE
        pltpu.sync_copy(cvec, cv)
        cval = cv[...]
        zval = jnp.zeros((16,), jnp.float32)
        lanes = lax.iota(jnp.int32, 16)

        idx_d = {}
        gat_d = {}
        out_d = {}

        def start_idx(j):
            b = j % NI
            off = base + j * K3
            idx_d[j] = (
                pltpu.async_copy(srcs.at[pl.ds(off, K3)], sidx[b], sem_i[b]),
                pltpu.async_copy(dsts.at[pl.ds(off, K3)], didx[b], sem_i[b]),
            )

        def start_gathers(j):
            b = j % 2
            gat_d[j] = (
                pltpu.async_copy(z.at[sidx[j % NI]], zs[b], sem_zs[b]),
                pltpu.async_copy(zw.at[didx[j % NI]], zd[b], sem_zd[b]),
            )

        start_idx(0)
        if nchunks > 1:
            start_idx(1)
        idx_d[0][0].wait()
        idx_d[0][1].wait()
        start_gathers(0)
        for j in range(nchunks):
            b = j % 2
            gat_d[j][0].wait()
            gat_d[j][1].wait()
            if j + 1 < nchunks:
                idx_d[j + 1][0].wait()
                idx_d[j + 1][1].wait()
                start_gathers(j + 1)
            if j + 2 < nchunks:
                start_idx(j + 2)
            if j >= 2:
                out_d[j - 2].wait()

            zsb = zs[b]
            zdb = zd[b]
            ovb = ov[b]

            def group(i, carry2):
                rows = i * 16 + lanes
                parts = [cval, zval, zval, zval]
                for d in range(16):
                    cols = jnp.full((16,), d, jnp.int32)
                    sv = plsc.load_gather(zsb, [rows, cols])
                    dv = plsc.load_gather(zdb, [rows, cols])
                    parts[d % 4] = parts[d % 4] + sv * dv
                acc = (parts[0] + parts[1]) + (parts[2] + parts[3])
                sig = 1.0 / (1.0 + jnp.exp(-acc))
                ovb[pl.ds(i * 16, 16)] = sig
                return carry2

            lax.fori_loop(0, ngroups, group, 0)
            out_d[j] = pltpu.async_copy(
                ovb, out.at[pl.ds(base + j * K3, K3)], sem_o[b])
        for j in range(max(0, nchunks - 2), nchunks):
            out_d[j].wait()

    return head


_sc_head = _make_sc_head()


def kernel(x, edge_index, Wl1, Wr1, b1, Wl2, Wr2, b2, W_fc1, b_fc1, W_fc2, b_fc2):
    src = edge_index[0]
    dst = edge_index[1]

    y1aug, hr = _tc_a(x, Wl1, Wr1, b1.reshape(1, 64))
    zrows1 = jnp.zeros((N_PER_TILE, DPAD), jnp.float32)
    acc1 = _sc_agg1(y1aug, src, dst, zrows1)

    y2, zr, inv = _tc_b(acc1, hr, Wl2, Wr2, b2.reshape(1, 16))
    zrows2 = jnp.zeros((N_PER_TILE, 16), jnp.float32)
    acc2 = _sc_agg2(y2, src, dst, zrows2)

    z, zw = _tc_c(acc2, inv, zr, W_fc2.T, W_fc1.T)

    cscalar = jnp.dot(b_fc1, W_fc2[:, 0]) + b_fc2[0]
    cvec = jnp.full((16,), cscalar, jnp.float32)
    out = _sc_head(z, zw, src, dst, cvec)
    return out.reshape(E, 1)


# speedup vs baseline: 17.4359x; 1.1565x over previous
import functools

import jax
import jax.numpy as jnp
from jax import lax
from jax.experimental import pallas as pl
from jax.experimental.pallas import tpu as pltpu
from jax.experimental.pallas import tpu_sc as plsc

N = 10000
NPAD = 10240
E = 320000
DPAD = 80

NC = 2
NS = 16
E_PER_TILE = E // (NC * NS)
N_PER_TILE = NPAD // NS

K1 = 200
K2 = 1000
K3 = 400


def _tc_a_body(x_ref, wl_ref, wr_ref, b_ref, y1aug_ref, hr_ref):
    xb = x_ref[...]
    y1 = jnp.dot(xb, wl_ref[...], preferred_element_type=jnp.float32)
    r = xb.shape[0]
    ones = jnp.ones((r, 1), jnp.float32)
    zeros = jnp.zeros((r, DPAD - 65), jnp.float32)
    y1aug_ref[...] = jnp.concatenate([y1, ones, zeros], axis=1)
    hr_ref[...] = jnp.dot(xb, wr_ref[...], preferred_element_type=jnp.float32) + b_ref[...]


def _tc_a(x, wl1, wr1, b1):
    blk = 1000
    grid = N // blk
    return pl.pallas_call(
        _tc_a_body,
        grid=(grid,),
        in_specs=[
            pl.BlockSpec((blk, 128), lambda i: (i, 0)),
            pl.BlockSpec((128, 64), lambda i: (0, 0)),
            pl.BlockSpec((128, 64), lambda i: (0, 0)),
            pl.BlockSpec((1, 64), lambda i: (0, 0)),
        ],
        out_specs=[
            pl.BlockSpec((blk, DPAD), lambda i: (i, 0)),
            pl.BlockSpec((blk, 64), lambda i: (i, 0)),
        ],
        out_shape=[
            jax.ShapeDtypeStruct((N, DPAD), jnp.float32),
            jax.ShapeDtypeStruct((N, 64), jnp.float32),
        ],
    )(x, wl1, wr1, b1)


def _tc_b_body(acc_ref, hr_ref, wl2_ref, wr2_ref, b2_ref, y2_ref, zr_ref, inv_ref):
    s = acc_ref[0] + acc_ref[1]
    cnt = s[:, 64:65]
    inv = 1.0 / jnp.maximum(cnt, 1.0)
    h = s[:, :64] * inv + hr_ref[...]
    y2_ref[...] = jnp.dot(h, wl2_ref[...], preferred_element_type=jnp.float32)
    zr_ref[...] = jnp.dot(h, wr2_ref[...], preferred_element_type=jnp.float32) + b2_ref[...]
    inv_ref[...] = inv


def _tc_b(acc1, hr, wl2, wr2, b2):
    blk = 1000
    grid = N // blk
    return pl.pallas_call(
        _tc_b_body,
        grid=(grid,),
        in_specs=[
            pl.BlockSpec((2, blk, DPAD), lambda i: (0, i, 0)),
            pl.BlockSpec((blk, 64), lambda i: (i, 0)),
            pl.BlockSpec((64, 16), lambda i: (0, 0)),
            pl.BlockSpec((64, 16), lambda i: (0, 0)),
            pl.BlockSpec((1, 16), lambda i: (0, 0)),
        ],
        out_specs=[
            pl.BlockSpec((blk, 16), lambda i: (i, 0)),
            pl.BlockSpec((blk, 16), lambda i: (i, 0)),
            pl.BlockSpec((blk, 1), lambda i: (i, 0)),
        ],
        out_shape=[
            jax.ShapeDtypeStruct((N, 16), jnp.float32),
            jax.ShapeDtypeStruct((N, 16), jnp.float32),
            jax.ShapeDtypeStruct((N, 1), jnp.float32),
        ],
    )(acc1, hr, wl2, wr2, b2)


def _tc_c_body(acc2_ref, inv_ref, zr_ref, wf2t_ref, wf1t_ref, z_ref, zw_ref):
    s = acc2_ref[0] + acc2_ref[1]
    z = s * inv_ref[...] + zr_ref[...]
    z_ref[...] = z
    wrow = jnp.dot(wf2t_ref[...], wf1t_ref[...], preferred_element_type=jnp.float32)
    zw_ref[...] = z * wrow


def _tc_c(acc2, inv, zr, wf2t, wf1t):
    blk = 1000
    grid = N // blk
    return pl.pallas_call(
        _tc_c_body,
        grid=(grid,),
        in_specs=[
            pl.BlockSpec((2, blk, 16), lambda i: (0, i, 0)),
            pl.BlockSpec((blk, 1), lambda i: (i, 0)),
            pl.BlockSpec((blk, 16), lambda i: (i, 0)),
            pl.BlockSpec((1, 8), lambda i: (0, 0)),
            pl.BlockSpec((8, 16), lambda i: (0, 0)),
        ],
        out_specs=[
            pl.BlockSpec((blk, 16), lambda i: (i, 0)),
            pl.BlockSpec((blk, 16), lambda i: (i, 0)),
        ],
        out_shape=[
            jax.ShapeDtypeStruct((N, 16), jnp.float32),
            jax.ShapeDtypeStruct((N, 16), jnp.float32),
        ],
    )(acc2, inv, zr, wf2t, wf1t)


def _make_sc_agg(dcols, kchunk):
    nchunks = E_PER_TILE // kchunk
    NB = 4
    NI = 6
    mesh = plsc.VectorSubcoreMesh(core_axis_name="c", subcore_axis_name="s")

    scratch = (
        [pltpu.VMEM((kchunk,), jnp.int32) for _ in range(NI)]
        + [pltpu.VMEM((kchunk,), jnp.int32) for _ in range(NI)]
        + [pltpu.VMEM((kchunk, dcols), jnp.float32) for _ in range(NB)]
        + [pltpu.VMEM_SHARED((NPAD, dcols), jnp.float32)]
        + [pltpu.SemaphoreType.DMA for _ in range(NI + 2 * NB)]
    )

    @functools.partial(
        pl.kernel,
        mesh=mesh,
        compiler_params=pltpu.CompilerParams(use_tc_tiling_on_sc=False),
        out_type=jax.ShapeDtypeStruct((NC, NPAD, dcols), jnp.float32),
        scratch_types=scratch,
    )
    def agg(table, srcs, dsts, zrows, acc_out, *sc):
        sidx = sc[0:NI]
        didx = sc[NI:2 * NI]
        rows = sc[2 * NI:2 * NI + NB]
        acc_sh = sc[2 * NI + NB]
        sem_i = sc[2 * NI + NB + 1:2 * NI + NB + 1 + NI]
        sem_g = sc[2 * NI + NB + 1 + NI:2 * NI + NB + 1 + NI + NB]
        sem_s = sc[2 * NI + NB + 1 + NI + NB:2 * NI + NB + 1 + NI + 2 * NB]

        c = lax.axis_index("c")
        s = lax.axis_index("s")
        pltpu.sync_copy(zrows, acc_sh.at[pl.ds(s * N_PER_TILE, N_PER_TILE)])
        plsc.subcore_barrier()
        base = c * (E // NC) + s * E_PER_TILE

        idx_d = {}
        gat_d = {}
        sca_d = {}

        def start_idx(j):
            b = j % NI
            off = base + j * kchunk
            idx_d[j] = (
                pltpu.async_copy(srcs.at[pl.ds(off, kchunk)], sidx[b], sem_i[b]),
                pltpu.async_copy(dsts.at[pl.ds(off, kchunk)], didx[b], sem_i[b]),
            )

        def start_gather(j):
            gat_d[j] = pltpu.async_copy(
                table.at[sidx[j % NI]], rows[j % NB], sem_g[j % NB])

        def start_scatter(j):
            sca_d[j] = pltpu.async_copy(
                rows[j % NB], acc_sh.at[didx[j % NI]], sem_s[j % NB], add=True)

        start_idx(0)
        if nchunks > 1:
            start_idx(1)
        idx_d[0][0].wait()
        idx_d[0][1].wait()
        start_gather(0)
        for j in range(nchunks):
            gat_d[j].wait()
            if j + 1 < nchunks:
                idx_d[j + 1][0].wait()
                idx_d[j + 1][1].wait()
                if j + 1 >= NB:
                    sca_d[j + 1 - NB].wait()
                start_gather(j + 1)
            start_scatter(j)
            if j + 2 < nchunks:
                start_idx(j + 2)
        for j in range(max(0, nchunks - NB), nchunks):
            sca_d[j].wait()

        plsc.subcore_barrier()
        pltpu.sync_copy(
            acc_sh.at[pl.ds(s * N_PER_TILE, N_PER_TILE)],
            acc_out.at[c, pl.ds(s * N_PER_TILE, N_PER_TILE)],
        )

    return agg


_sc_agg1 = _make_sc_agg(DPAD, K1)
_sc_agg2 = _make_sc_agg(16, K2)


def _make_sc_head():
    nchunks = E_PER_TILE // K3
    ngroups = K3 // 16
    NI = 3
    mesh = plsc.VectorSubcoreMesh(core_axis_name="c", subcore_axis_name="s")

    scratch = (
        [pltpu.VMEM((K3,), jnp.int32) for _ in range(NI)]
        + [pltpu.VMEM((K3,), jnp.int32) for _ in range(NI)]
        + [pltpu.VMEM((K3, 16), jnp.float32) for _ in range(2)]
        + [pltpu.VMEM((K3, 16), jnp.float32) for _ in range(2)]
        + [pltpu.VMEM((K3,), jnp.float32) for _ in range(2)]
        + [pltpu.VMEM((16,), jnp.float32)]
        + [pltpu.SemaphoreType.DMA for _ in range(NI + 6)]
    )

    @functools.partial(
        pl.kernel,
        mesh=mesh,
        compiler_params=pltpu.CompilerParams(
            use_tc_tiling_on_sc=False, needs_layout_passes=False),
        out_type=jax.ShapeDtypeStruct((E,), jnp.float32),
        scratch_types=scratch,
    )
    def head(z, zw, srcs, dsts, cvec, out, *sc):
        sidx = sc[0:NI]
        didx = sc[NI:2 * NI]
        zs = sc[2 * NI:2 * NI + 2]
        zd = sc[2 * NI + 2:2 * NI + 4]
        ov = sc[2 * NI + 4:2 * NI + 6]
        cv = sc[2 * NI + 6]
        sem_i = sc[2 * NI + 7:2 * NI + 7 + NI]
        sem_zs = sc[2 * NI + 7 + NI:2 * NI + 9 + NI]
        sem_zd = sc[2 * NI + 9 + NI:2 * NI + 11 + NI]
        sem_o = sc[2 * NI + 11 + NI:2 * NI + 13 + NI]

        c = lax.axis_index("c")
        s = lax.axis_index("s")
        base = c * (E // NC) + s * E_PER_TILE
        pltpu.sync_copy(cvec, cv)
        cval = cv[...]
        zval = jnp.zeros((16,), jnp.float32)
        lanes = lax.iota(jnp.int32, 16)

        idx_d = {}
        gat_d = {}
        out_d = {}

        def start_idx(j):
            b = j % NI
            off = base + j * K3
            idx_d[j] = (
                pltpu.async_copy(srcs.at[pl.ds(off, K3)], sidx[b], sem_i[b]),
                pltpu.async_copy(dsts.at[pl.ds(off, K3)], didx[b], sem_i[b]),
            )

        def start_gathers(j):
            b = j % 2
            gat_d[j] = (
                pltpu.async_copy(z.at[sidx[j % NI]], zs[b], sem_zs[b]),
                pltpu.async_copy(zw.at[didx[j % NI]], zd[b], sem_zd[b]),
            )

        start_idx(0)
        if nchunks > 1:
            start_idx(1)
        idx_d[0][0].wait()
        idx_d[0][1].wait()
        start_gathers(0)
        for j in range(nchunks):
            b = j % 2
            gat_d[j][0].wait()
            gat_d[j][1].wait()
            if j + 1 < nchunks:
                idx_d[j + 1][0].wait()
                idx_d[j + 1][1].wait()
                start_gathers(j + 1)
            if j + 2 < nchunks:
                start_idx(j + 2)
            if j >= 2:
                out_d[j - 2].wait()

            zsb = zs[b]
            zdb = zd[b]
            ovb = ov[b]

            def group(i, carry2):
                rows = i * 16 + lanes
                parts = [cval, zval, zval, zval]
                for d in range(16):
                    cols = (lanes + d) & 15
                    sv = plsc.load_gather(zsb, [rows, cols])
                    dv = plsc.load_gather(zdb, [rows, cols])
                    parts[d % 4] = parts[d % 4] + sv * dv
                acc = (parts[0] + parts[1]) + (parts[2] + parts[3])
                sig = 1.0 / (1.0 + jnp.exp(-acc))
                ovb[pl.ds(i * 16, 16)] = sig
                return carry2

            lax.fori_loop(0, ngroups, group, 0)
            out_d[j] = pltpu.async_copy(
                ovb, out.at[pl.ds(base + j * K3, K3)], sem_o[b])
        for j in range(max(0, nchunks - 2), nchunks):
            out_d[j].wait()

    return head


_sc_head = _make_sc_head()


def kernel(x, edge_index, Wl1, Wr1, b1, Wl2, Wr2, b2, W_fc1, b_fc1, W_fc2, b_fc2):
    src = edge_index[0]
    dst = edge_index[1]

    y1aug, hr = _tc_a(x, Wl1, Wr1, b1.reshape(1, 64))
    zrows1 = jnp.zeros((N_PER_TILE, DPAD), jnp.float32)
    acc1 = _sc_agg1(y1aug, src, dst, zrows1)

    y2, zr, inv = _tc_b(acc1, hr, Wl2, Wr2, b2.reshape(1, 16))
    zrows2 = jnp.zeros((N_PER_TILE, 16), jnp.float32)
    acc2 = _sc_agg2(y2, src, dst, zrows2)

    z, zw = _tc_c(acc2, inv, zr, W_fc2.T, W_fc1.T)

    cscalar = jnp.dot(b_fc1, W_fc2[:, 0]) + b_fc2[0]
    cvec = jnp.full((16,), cscalar, jnp.float32)
    out = _sc_head(z, zw, src, dst, cvec)
    return out.reshape(E, 1)


# speedup vs baseline: 17.5378x; 1.0058x over previous
import functools

import jax
import jax.numpy as jnp
from jax import lax
from jax.experimental import pallas as pl
from jax.experimental.pallas import tpu as pltpu
from jax.experimental.pallas import tpu_sc as plsc

N = 10000
NPAD = 10240
E = 320000
DPAD = 72

NC = 2
NS = 16
E_PER_TILE = E // (NC * NS)
N_PER_TILE = NPAD // NS

K1 = 200
K2 = 1000
K3 = 400


def _tc_a_body(x_ref, wl_ref, wr_ref, b_ref, y1aug_ref, hr_ref):
    xb = x_ref[...]
    y1 = jnp.dot(xb, wl_ref[...], preferred_element_type=jnp.float32)
    r = xb.shape[0]
    ones = jnp.ones((r, 1), jnp.float32)
    zeros = jnp.zeros((r, DPAD - 65), jnp.float32)
    y1aug_ref[...] = jnp.concatenate([y1, ones, zeros], axis=1)
    hr_ref[...] = jnp.dot(xb, wr_ref[...], preferred_element_type=jnp.float32) + b_ref[...]


def _tc_a(x, wl1, wr1, b1):
    blk = 1000
    grid = N // blk
    return pl.pallas_call(
        _tc_a_body,
        grid=(grid,),
        in_specs=[
            pl.BlockSpec((blk, 128), lambda i: (i, 0)),
            pl.BlockSpec((128, 64), lambda i: (0, 0)),
            pl.BlockSpec((128, 64), lambda i: (0, 0)),
            pl.BlockSpec((1, 64), lambda i: (0, 0)),
        ],
        out_specs=[
            pl.BlockSpec((blk, DPAD), lambda i: (i, 0)),
            pl.BlockSpec((blk, 64), lambda i: (i, 0)),
        ],
        out_shape=[
            jax.ShapeDtypeStruct((N, DPAD), jnp.float32),
            jax.ShapeDtypeStruct((N, 64), jnp.float32),
        ],
    )(x, wl1, wr1, b1)


def _tc_b_body(acc_ref, hr_ref, wl2_ref, wr2_ref, b2_ref, y2_ref, zr_ref, inv_ref):
    s = acc_ref[0] + acc_ref[1]
    cnt = s[:, 64:65]
    inv = 1.0 / jnp.maximum(cnt, 1.0)
    h = s[:, :64] * inv + hr_ref[...]
    y2_ref[...] = jnp.dot(h, wl2_ref[...], preferred_element_type=jnp.float32)
    zr_ref[...] = jnp.dot(h, wr2_ref[...], preferred_element_type=jnp.float32) + b2_ref[...]
    inv_ref[...] = inv


def _tc_b(acc1, hr, wl2, wr2, b2):
    blk = 1000
    grid = N // blk
    return pl.pallas_call(
        _tc_b_body,
        grid=(grid,),
        in_specs=[
            pl.BlockSpec((2, blk, DPAD), lambda i: (0, i, 0)),
            pl.BlockSpec((blk, 64), lambda i: (i, 0)),
            pl.BlockSpec((64, 16), lambda i: (0, 0)),
            pl.BlockSpec((64, 16), lambda i: (0, 0)),
            pl.BlockSpec((1, 16), lambda i: (0, 0)),
        ],
        out_specs=[
            pl.BlockSpec((blk, 16), lambda i: (i, 0)),
            pl.BlockSpec((blk, 16), lambda i: (i, 0)),
            pl.BlockSpec((blk, 1), lambda i: (i, 0)),
        ],
        out_shape=[
            jax.ShapeDtypeStruct((N, 16), jnp.float32),
            jax.ShapeDtypeStruct((N, 16), jnp.float32),
            jax.ShapeDtypeStruct((N, 1), jnp.float32),
        ],
    )(acc1, hr, wl2, wr2, b2)


def _tc_c_body(acc2_ref, inv_ref, zr_ref, wf2t_ref, wf1t_ref, z_ref, zw_ref):
    s = acc2_ref[0] + acc2_ref[1]
    z = s * inv_ref[...] + zr_ref[...]
    z_ref[...] = z
    wrow = jnp.dot(wf2t_ref[...], wf1t_ref[...], preferred_element_type=jnp.float32)
    zw_ref[...] = z * wrow


def _tc_c(acc2, inv, zr, wf2t, wf1t):
    blk = 1000
    grid = N // blk
    return pl.pallas_call(
        _tc_c_body,
        grid=(grid,),
        in_specs=[
            pl.BlockSpec((2, blk, 16), lambda i: (0, i, 0)),
            pl.BlockSpec((blk, 1), lambda i: (i, 0)),
            pl.BlockSpec((blk, 16), lambda i: (i, 0)),
            pl.BlockSpec((1, 8), lambda i: (0, 0)),
            pl.BlockSpec((8, 16), lambda i: (0, 0)),
        ],
        out_specs=[
            pl.BlockSpec((blk, 16), lambda i: (i, 0)),
            pl.BlockSpec((blk, 16), lambda i: (i, 0)),
        ],
        out_shape=[
            jax.ShapeDtypeStruct((N, 16), jnp.float32),
            jax.ShapeDtypeStruct((N, 16), jnp.float32),
        ],
    )(acc2, inv, zr, wf2t, wf1t)


def _make_sc_agg(dcols, kchunk):
    nchunks = E_PER_TILE // kchunk
    NB = 4
    NI = 6
    mesh = plsc.VectorSubcoreMesh(core_axis_name="c", subcore_axis_name="s")

    scratch = (
        [pltpu.VMEM((kchunk,), jnp.int32) for _ in range(NI)]
        + [pltpu.VMEM((kchunk,), jnp.int32) for _ in range(NI)]
        + [pltpu.VMEM((kchunk, dcols), jnp.float32) for _ in range(NB)]
        + [pltpu.VMEM_SHARED((NPAD, dcols), jnp.float32)]
        + [pltpu.SemaphoreType.DMA for _ in range(NI + 2 * NB)]
    )

    @functools.partial(
        pl.kernel,
        mesh=mesh,
        compiler_params=pltpu.CompilerParams(use_tc_tiling_on_sc=False),
        out_type=jax.ShapeDtypeStruct((NC, NPAD, dcols), jnp.float32),
        scratch_types=scratch,
    )
    def agg(table, srcs, dsts, zrows, acc_out, *sc):
        sidx = sc[0:NI]
        didx = sc[NI:2 * NI]
        rows = sc[2 * NI:2 * NI + NB]
        acc_sh = sc[2 * NI + NB]
        sem_i = sc[2 * NI + NB + 1:2 * NI + NB + 1 + NI]
        sem_g = sc[2 * NI + NB + 1 + NI:2 * NI + NB + 1 + NI + NB]
        sem_s = sc[2 * NI + NB + 1 + NI + NB:2 * NI + NB + 1 + NI + 2 * NB]

        c = lax.axis_index("c")
        s = lax.axis_index("s")
        pltpu.sync_copy(zrows, acc_sh.at[pl.ds(s * N_PER_TILE, N_PER_TILE)])
        plsc.subcore_barrier()
        base = c * (E // NC) + s * E_PER_TILE

        idx_d = {}
        gat_d = {}
        sca_d = {}

        def start_idx(j):
            b = j % NI
            off = base + j * kchunk
            idx_d[j] = (
                pltpu.async_copy(srcs.at[pl.ds(off, kchunk)], sidx[b], sem_i[b]),
                pltpu.async_copy(dsts.at[pl.ds(off, kchunk)], didx[b], sem_i[b]),
            )

        def start_gather(j):
            gat_d[j] = pltpu.async_copy(
                table.at[sidx[j % NI]], rows[j % NB], sem_g[j % NB])

        def start_scatter(j):
            sca_d[j] = pltpu.async_copy(
                rows[j % NB], acc_sh.at[didx[j % NI]], sem_s[j % NB], add=True)

        start_idx(0)
        if nchunks > 1:
            start_idx(1)
        idx_d[0][0].wait()
        idx_d[0][1].wait()
        start_gather(0)
        for j in range(nchunks):
            gat_d[j].wait()
            if j + 1 < nchunks:
                idx_d[j + 1][0].wait()
                idx_d[j + 1][1].wait()
                if j + 1 >= NB:
                    sca_d[j + 1 - NB].wait()
                start_gather(j + 1)
            start_scatter(j)
            if j + 2 < nchunks:
                start_idx(j + 2)
        for j in range(max(0, nchunks - NB), nchunks):
            sca_d[j].wait()

        plsc.subcore_barrier()
        pltpu.sync_copy(
            acc_sh.at[pl.ds(s * N_PER_TILE, N_PER_TILE)],
            acc_out.at[c, pl.ds(s * N_PER_TILE, N_PER_TILE)],
        )

    return agg


_sc_agg1 = _make_sc_agg(DPAD, K1)
_sc_agg2 = _make_sc_agg(16, K2)


def _make_sc_head():
    nchunks = E_PER_TILE // K3
    ngroups = K3 // 16
    NI = 3
    mesh = plsc.VectorSubcoreMesh(core_axis_name="c", subcore_axis_name="s")

    scratch = (
        [pltpu.VMEM((K3,), jnp.int32) for _ in range(NI)]
        + [pltpu.VMEM((K3,), jnp.int32) for _ in range(NI)]
        + [pltpu.VMEM((K3, 16), jnp.float32) for _ in range(2)]
        + [pltpu.VMEM((K3, 16), jnp.float32) for _ in range(2)]
        + [pltpu.VMEM((K3,), jnp.float32) for _ in range(2)]
        + [pltpu.VMEM((16,), jnp.float32)]
        + [pltpu.SemaphoreType.DMA for _ in range(NI + 6)]
    )

    @functools.partial(
        pl.kernel,
        mesh=mesh,
        compiler_params=pltpu.CompilerParams(
            use_tc_tiling_on_sc=False, needs_layout_passes=False),
        out_type=jax.ShapeDtypeStruct((E,), jnp.float32),
        scratch_types=scratch,
    )
    def head(z, zw, srcs, dsts, cvec, out, *sc):
        sidx = sc[0:NI]
        didx = sc[NI:2 * NI]
        zs = sc[2 * NI:2 * NI + 2]
        zd = sc[2 * NI + 2:2 * NI + 4]
        ov = sc[2 * NI + 4:2 * NI + 6]
        cv = sc[2 * NI + 6]
        sem_i = sc[2 * NI + 7:2 * NI + 7 + NI]
        sem_zs = sc[2 * NI + 7 + NI:2 * NI + 9 + NI]
        sem_zd = sc[2 * NI + 9 + NI:2 * NI + 11 + NI]
        sem_o = sc[2 * NI + 11 + NI:2 * NI + 13 + NI]

        c = lax.axis_index("c")
        s = lax.axis_index("s")
        base = c * (E // NC) + s * E_PER_TILE
        pltpu.sync_copy(cvec, cv)
        cval = cv[...]
        zval = jnp.zeros((16,), jnp.float32)
        lanes = lax.iota(jnp.int32, 16)

        idx_d = {}
        gat_d = {}
        out_d = {}

        def start_idx(j):
            b = j % NI
            off = base + j * K3
            idx_d[j] = (
                pltpu.async_copy(srcs.at[pl.ds(off, K3)], sidx[b], sem_i[b]),
                pltpu.async_copy(dsts.at[pl.ds(off, K3)], didx[b], sem_i[b]),
            )

        def start_gathers(j):
            b = j % 2
            gat_d[j] = (
                pltpu.async_copy(z.at[sidx[j % NI]], zs[b], sem_zs[b]),
                pltpu.async_copy(zw.at[didx[j % NI]], zd[b], sem_zd[b]),
            )

        start_idx(0)
        if nchunks > 1:
            start_idx(1)
        idx_d[0][0].wait()
        idx_d[0][1].wait()
        start_gathers(0)
        for j in range(nchunks):
            b = j % 2
            gat_d[j][0].wait()
            gat_d[j][1].wait()
            if j + 1 < nchunks:
                idx_d[j + 1][0].wait()
                idx_d[j + 1][1].wait()
                start_gathers(j + 1)
            if j + 2 < nchunks:
                start_idx(j + 2)
            if j >= 2:
                out_d[j - 2].wait()

            zsb = zs[b]
            zdb = zd[b]
            ovb = ov[b]

            def group(i, carry2):
                rows = i * 16 + lanes
                parts = [cval, zval, zval, zval]
                for d in range(16):
                    cols = (lanes + d) & 15
                    sv = plsc.load_gather(zsb, [rows, cols])
                    dv = plsc.load_gather(zdb, [rows, cols])
                    parts[d % 4] = parts[d % 4] + sv * dv
                acc = (parts[0] + parts[1]) + (parts[2] + parts[3])
                sig = 1.0 / (1.0 + jnp.exp(-acc))
                ovb[pl.ds(i * 16, 16)] = sig
                return carry2

            lax.fori_loop(0, ngroups, group, 0)
            out_d[j] = pltpu.async_copy(
                ovb, out.at[pl.ds(base + j * K3, K3)], sem_o[b])
        for j in range(max(0, nchunks - 2), nchunks):
            out_d[j].wait()

    return head


_sc_head = _make_sc_head()


def kernel(x, edge_index, Wl1, Wr1, b1, Wl2, Wr2, b2, W_fc1, b_fc1, W_fc2, b_fc2):
    src = edge_index[0]
    dst = edge_index[1]

    y1aug, hr = _tc_a(x, Wl1, Wr1, b1.reshape(1, 64))
    zrows1 = jnp.zeros((N_PER_TILE, DPAD), jnp.float32)
    acc1 = _sc_agg1(y1aug, src, dst, zrows1)

    y2, zr, inv = _tc_b(acc1, hr, Wl2, Wr2, b2.reshape(1, 16))
    zrows2 = jnp.zeros((N_PER_TILE, 16), jnp.float32)
    acc2 = _sc_agg2(y2, src, dst, zrows2)

    z, zw = _tc_c(acc2, inv, zr, W_fc2.T, W_fc1.T)

    cscalar = jnp.dot(b_fc1, W_fc2[:, 0]) + b_fc2[0]
    cvec = jnp.full((16,), cscalar, jnp.float32)
    out = _sc_head(z, zw, src, dst, cvec)
    return out.reshape(E, 1)


# speedup vs baseline: 17.7598x; 1.0127x over previous
import functools

import jax
import jax.numpy as jnp
from jax import lax
from jax.experimental import pallas as pl
from jax.experimental.pallas import tpu as pltpu
from jax.experimental.pallas import tpu_sc as plsc

N = 10000
NPAD = 10240
E = 320000
DPAD = 64

NC = 2
NS = 16
E_PER_TILE = E // (NC * NS)
N_PER_TILE = NPAD // NS

K1 = 200
K2 = 1000
K3 = 400


def _tc_a_body(x_ref, wl_ref, wr_ref, b_ref, y1aug_ref, hr_ref):
    xb = x_ref[...]
    y1 = jnp.dot(xb, wl_ref[...], preferred_element_type=jnp.float32)
    r = xb.shape[0]
    y1aug_ref[...] = y1
    hr_ref[...] = jnp.dot(xb, wr_ref[...], preferred_element_type=jnp.float32) + b_ref[...]


def _tc_a(x, wl1, wr1, b1):
    blk = 1000
    grid = N // blk
    return pl.pallas_call(
        _tc_a_body,
        grid=(grid,),
        in_specs=[
            pl.BlockSpec((blk, 128), lambda i: (i, 0)),
            pl.BlockSpec((128, 64), lambda i: (0, 0)),
            pl.BlockSpec((128, 64), lambda i: (0, 0)),
            pl.BlockSpec((1, 64), lambda i: (0, 0)),
        ],
        out_specs=[
            pl.BlockSpec((blk, DPAD), lambda i: (i, 0)),
            pl.BlockSpec((blk, 64), lambda i: (i, 0)),
        ],
        out_shape=[
            jax.ShapeDtypeStruct((N, DPAD), jnp.float32),
            jax.ShapeDtypeStruct((N, 64), jnp.float32),
        ],
    )(x, wl1, wr1, b1)


def _tc_b_body(acc_ref, cnt_ref, hr_ref, wl2_ref, wr2_ref, b2_ref, y2_ref, zr_ref, inv_ref):
    s = acc_ref[0] + acc_ref[1]
    cnt = cnt_ref[0] + cnt_ref[1]
    inv = 1.0 / jnp.maximum(cnt, 1.0)
    h = s * inv + hr_ref[...]
    y2_ref[...] = jnp.dot(h, wl2_ref[...], preferred_element_type=jnp.float32)
    zr_ref[...] = jnp.dot(h, wr2_ref[...], preferred_element_type=jnp.float32) + b2_ref[...]
    inv_ref[...] = inv


def _tc_b(acc1, cnt, hr, wl2, wr2, b2):
    blk = 1000
    grid = N // blk
    return pl.pallas_call(
        _tc_b_body,
        grid=(grid,),
        in_specs=[
            pl.BlockSpec((2, blk, DPAD), lambda i: (0, i, 0)),
            pl.BlockSpec((2, blk, 1), lambda i: (0, i, 0)),
            pl.BlockSpec((blk, 64), lambda i: (i, 0)),
            pl.BlockSpec((64, 16), lambda i: (0, 0)),
            pl.BlockSpec((64, 16), lambda i: (0, 0)),
            pl.BlockSpec((1, 16), lambda i: (0, 0)),
        ],
        out_specs=[
            pl.BlockSpec((blk, 16), lambda i: (i, 0)),
            pl.BlockSpec((blk, 16), lambda i: (i, 0)),
            pl.BlockSpec((blk, 1), lambda i: (i, 0)),
        ],
        out_shape=[
            jax.ShapeDtypeStruct((N, 16), jnp.float32),
            jax.ShapeDtypeStruct((N, 16), jnp.float32),
            jax.ShapeDtypeStruct((N, 1), jnp.float32),
        ],
    )(acc1, cnt, hr, wl2, wr2, b2)


def _tc_c_body(acc2_ref, inv_ref, zr_ref, wf2t_ref, wf1t_ref, z_ref, zw_ref):
    s = acc2_ref[0] + acc2_ref[1]
    z = s * inv_ref[...] + zr_ref[...]
    z_ref[...] = z
    wrow = jnp.dot(wf2t_ref[...], wf1t_ref[...], preferred_element_type=jnp.float32)
    zw_ref[...] = z * wrow


def _tc_c(acc2, inv, zr, wf2t, wf1t):
    blk = 1000
    grid = N // blk
    return pl.pallas_call(
        _tc_c_body,
        grid=(grid,),
        in_specs=[
            pl.BlockSpec((2, blk, 16), lambda i: (0, i, 0)),
            pl.BlockSpec((blk, 1), lambda i: (i, 0)),
            pl.BlockSpec((blk, 16), lambda i: (i, 0)),
            pl.BlockSpec((1, 8), lambda i: (0, 0)),
            pl.BlockSpec((8, 16), lambda i: (0, 0)),
        ],
        out_specs=[
            pl.BlockSpec((blk, 16), lambda i: (i, 0)),
            pl.BlockSpec((blk, 16), lambda i: (i, 0)),
        ],
        out_shape=[
            jax.ShapeDtypeStruct((N, 16), jnp.float32),
            jax.ShapeDtypeStruct((N, 16), jnp.float32),
        ],
    )(acc2, inv, zr, wf2t, wf1t)


def _make_sc_agg(dcols, kchunk, with_cnt):
    nchunks = E_PER_TILE // kchunk
    NB = 4
    NI = 6
    mesh = plsc.VectorSubcoreMesh(core_axis_name="c", subcore_axis_name="s")

    scratch = (
        [pltpu.VMEM((kchunk,), jnp.int32) for _ in range(NI)]
        + [pltpu.VMEM((kchunk,), jnp.int32) for _ in range(NI)]
        + [pltpu.VMEM((kchunk, dcols), jnp.float32) for _ in range(NB)]
        + [pltpu.VMEM_SHARED((NPAD, dcols), jnp.float32)]
        + [pltpu.SemaphoreType.DMA for _ in range(NI + 2 * NB)]
    )
    out_type = [jax.ShapeDtypeStruct((NC, NPAD, dcols), jnp.float32)]
    if with_cnt:
        scratch += (
            [pltpu.VMEM(((kchunk + 15) // 16 * 16,), jnp.float32)]
            + [pltpu.VMEM_SHARED((NPAD,), jnp.float32)]
            + [pltpu.SemaphoreType.DMA for _ in range(NB)]
        )
        out_type.append(jax.ShapeDtypeStruct((NC, NPAD), jnp.float32))

    @functools.partial(
        pl.kernel,
        mesh=mesh,
        compiler_params=pltpu.CompilerParams(use_tc_tiling_on_sc=False),
        out_type=out_type,
        scratch_types=scratch,
    )
    def agg(table, srcs, dsts, zrows, zc, *outs_sc):
        if with_cnt:
            acc_out, cnt_out = outs_sc[0], outs_sc[1]
            sc = outs_sc[2:]
        else:
            acc_out = outs_sc[0]
            sc = outs_sc[1:]
        sidx = sc[0:NI]
        didx = sc[NI:2 * NI]
        rows = sc[2 * NI:2 * NI + NB]
        acc_sh = sc[2 * NI + NB]
        o = 2 * NI + NB + 1
        sem_i = sc[o:o + NI]
        sem_g = sc[o + NI:o + NI + NB]
        sem_s = sc[o + NI + NB:o + NI + 2 * NB]
        if with_cnt:
            o2 = o + NI + 2 * NB
            ones_v = sc[o2]
            cnt_sh = sc[o2 + 1]
            sem_c = sc[o2 + 2:o2 + 2 + NB]

        c = lax.axis_index("c")
        s = lax.axis_index("s")
        pltpu.sync_copy(zrows, acc_sh.at[pl.ds(s * N_PER_TILE, N_PER_TILE)])
        if with_cnt:
            def fill(i, carry):
                ones_v[pl.ds(i * 16, 16)] = jnp.ones((16,), jnp.float32)
                return carry
            lax.fori_loop(0, (kchunk + 15) // 16, fill, 0)
            pltpu.sync_copy(zc, cnt_sh.at[pl.ds(s * N_PER_TILE, N_PER_TILE)])
        plsc.subcore_barrier()
        base = c * (E // NC) + s * E_PER_TILE

        idx_d = {}
        gat_d = {}
        sca_d = {}
        cnt_d = {}

        def start_idx(j):
            b = j % NI
            off = base + j * kchunk
            idx_d[j] = (
                pltpu.async_copy(srcs.at[pl.ds(off, kchunk)], sidx[b], sem_i[b]),
                pltpu.async_copy(dsts.at[pl.ds(off, kchunk)], didx[b], sem_i[b]),
            )

        def start_gather(j):
            gat_d[j] = pltpu.async_copy(
                table.at[sidx[j % NI]], rows[j % NB], sem_g[j % NB])

        def start_scatter(j):
            sca_d[j] = pltpu.async_copy(
                rows[j % NB], acc_sh.at[didx[j % NI]], sem_s[j % NB], add=True)
            if with_cnt:
                cnt_d[j] = pltpu.async_copy(
                    ones_v.at[pl.ds(0, kchunk)], cnt_sh.at[didx[j % NI]],
                    sem_c[j % NB], add=True)

        start_idx(0)
        if nchunks > 1:
            start_idx(1)
        idx_d[0][0].wait()
        idx_d[0][1].wait()
        start_gather(0)
        for j in range(nchunks):
            gat_d[j].wait()
            if j + 1 < nchunks:
                idx_d[j + 1][0].wait()
                idx_d[j + 1][1].wait()
                if j + 1 >= NB:
                    sca_d[j + 1 - NB].wait()
                    if with_cnt:
                        cnt_d[j + 1 - NB].wait()
                start_gather(j + 1)
            start_scatter(j)
            if j + 2 < nchunks:
                start_idx(j + 2)
        for j in range(max(0, nchunks - NB), nchunks):
            sca_d[j].wait()
            if with_cnt:
                cnt_d[j].wait()

        plsc.subcore_barrier()
        pltpu.sync_copy(
            acc_sh.at[pl.ds(s * N_PER_TILE, N_PER_TILE)],
            acc_out.at[c, pl.ds(s * N_PER_TILE, N_PER_TILE)],
        )
        if with_cnt:
            pltpu.sync_copy(
                cnt_sh.at[pl.ds(s * N_PER_TILE, N_PER_TILE)],
                cnt_out.at[c, pl.ds(s * N_PER_TILE, N_PER_TILE)],
            )

    return agg


_sc_agg1 = _make_sc_agg(DPAD, K1, with_cnt=True)
_sc_agg2 = _make_sc_agg(16, K2, with_cnt=False)


def _make_sc_head():
    nchunks = E_PER_TILE // K3
    ngroups = K3 // 16
    NI = 3
    mesh = plsc.VectorSubcoreMesh(core_axis_name="c", subcore_axis_name="s")

    scratch = (
        [pltpu.VMEM((K3,), jnp.int32) for _ in range(NI)]
        + [pltpu.VMEM((K3,), jnp.int32) for _ in range(NI)]
        + [pltpu.VMEM((K3, 16), jnp.float32) for _ in range(2)]
        + [pltpu.VMEM((K3, 16), jnp.float32) for _ in range(2)]
        + [pltpu.VMEM((K3,), jnp.float32) for _ in range(2)]
        + [pltpu.VMEM((16,), jnp.float32)]
        + [pltpu.SemaphoreType.DMA for _ in range(NI + 6)]
    )

    @functools.partial(
        pl.kernel,
        mesh=mesh,
        compiler_params=pltpu.CompilerParams(
            use_tc_tiling_on_sc=False, needs_layout_passes=False),
        out_type=jax.ShapeDtypeStruct((E,), jnp.float32),
        scratch_types=scratch,
    )
    def head(z, zw, srcs, dsts, cvec, out, *sc):
        sidx = sc[0:NI]
        didx = sc[NI:2 * NI]
        zs = sc[2 * NI:2 * NI + 2]
        zd = sc[2 * NI + 2:2 * NI + 4]
        ov = sc[2 * NI + 4:2 * NI + 6]
        cv = sc[2 * NI + 6]
        sem_i = sc[2 * NI + 7:2 * NI + 7 + NI]
        sem_zs = sc[2 * NI + 7 + NI:2 * NI + 9 + NI]
        sem_zd = sc[2 * NI + 9 + NI:2 * NI + 11 + NI]
        sem_o = sc[2 * NI + 11 + NI:2 * NI + 13 + NI]

        c = lax.axis_index("c")
        s = lax.axis_index("s")
        base = c * (E // NC) + s * E_PER_TILE
        pltpu.sync_copy(cvec, cv)
        cval = cv[...]
        zval = jnp.zeros((16,), jnp.float32)
        lanes = lax.iota(jnp.int32, 16)

        idx_d = {}
        gat_d = {}
        out_d = {}

        def start_idx(j):
            b = j % NI
            off = base + j * K3
            idx_d[j] = (
                pltpu.async_copy(srcs.at[pl.ds(off, K3)], sidx[b], sem_i[b]),
                pltpu.async_copy(dsts.at[pl.ds(off, K3)], didx[b], sem_i[b]),
            )

        def start_gathers(j):
            b = j % 2
            gat_d[j] = (
                pltpu.async_copy(z.at[sidx[j % NI]], zs[b], sem_zs[b]),
                pltpu.async_copy(zw.at[didx[j % NI]], zd[b], sem_zd[b]),
            )

        start_idx(0)
        if nchunks > 1:
            start_idx(1)
        idx_d[0][0].wait()
        idx_d[0][1].wait()
        start_gathers(0)
        for j in range(nchunks):
            b = j % 2
            gat_d[j][0].wait()
            gat_d[j][1].wait()
            if j + 1 < nchunks:
                idx_d[j + 1][0].wait()
                idx_d[j + 1][1].wait()
                start_gathers(j + 1)
            if j + 2 < nchunks:
                start_idx(j + 2)
            if j >= 2:
                out_d[j - 2].wait()

            zsb = zs[b]
            zdb = zd[b]
            ovb = ov[b]

            def group(i, carry2):
                rows = i * 16 + lanes
                parts = [cval, zval, zval, zval]
                for d in range(16):
                    cols = (lanes + d) & 15
                    sv = plsc.load_gather(zsb, [rows, cols])
                    dv = plsc.load_gather(zdb, [rows, cols])
                    parts[d % 4] = parts[d % 4] + sv * dv
                acc = (parts[0] + parts[1]) + (parts[2] + parts[3])
                sig = 1.0 / (1.0 + jnp.exp(-acc))
                ovb[pl.ds(i * 16, 16)] = sig
                return carry2

            lax.fori_loop(0, ngroups, group, 0)
            out_d[j] = pltpu.async_copy(
                ovb, out.at[pl.ds(base + j * K3, K3)], sem_o[b])
        for j in range(max(0, nchunks - 2), nchunks):
            out_d[j].wait()

    return head


_sc_head = _make_sc_head()


def kernel(x, edge_index, Wl1, Wr1, b1, Wl2, Wr2, b2, W_fc1, b_fc1, W_fc2, b_fc2):
    src = edge_index[0]
    dst = edge_index[1]

    y1aug, hr = _tc_a(x, Wl1, Wr1, b1.reshape(1, 64))
    zrows1 = jnp.zeros((N_PER_TILE, DPAD), jnp.float32)
    zc = jnp.zeros((N_PER_TILE,), jnp.float32)
    acc1, cnt = _sc_agg1(y1aug, src, dst, zrows1, zc)

    y2, zr, inv = _tc_b(acc1, cnt.reshape(NC, NPAD, 1), hr, Wl2, Wr2, b2.reshape(1, 16))
    zrows2 = jnp.zeros((N_PER_TILE, 16), jnp.float32)
    acc2, = _sc_agg2(y2, src, dst, zrows2, zc)

    z, zw = _tc_c(acc2, inv, zr, W_fc2.T, W_fc1.T)

    cscalar = jnp.dot(b_fc1, W_fc2[:, 0]) + b_fc2[0]
    cvec = jnp.full((16,), cscalar, jnp.float32)
    out = _sc_head(z, zw, src, dst, cvec)
    return out.reshape(E, 1)
